# Initial kernel scaffold; baseline (speedup 1.0000x reference)
import jax
import jax.numpy as jnp
from jax.experimental import pallas as pl


def kernel(h, edge_index, W, b):
    raise NotImplementedError("write your pallas kernel here")


# speedup vs baseline: 4.3869x; 4.3869x over previous
import functools

import jax
import jax.numpy as jnp
from jax import lax
from jax.experimental import pallas as pl
from jax.experimental.pallas import tpu as pltpu
from jax.experimental.pallas import tpu_sc as plsc

NC = 2
NS = 16
CH = 128


def _sc_aggregate(h_aug, e, z):
    n_pad, da = h_aug.shape
    nw = NC * NS
    e_pad = e.shape[1]
    epw = e_pad // nw
    k_chunks = epw // CH
    rpt = n_pad // NS

    mesh = plsc.VectorSubcoreMesh(core_axis_name="c", subcore_axis_name="s")

    @functools.partial(
        pl.kernel,
        out_type=jax.ShapeDtypeStruct((NC, n_pad, da), jnp.float32),
        mesh=mesh,
        scratch_types=[
            pltpu.VMEM((2, CH), jnp.int32),
            pltpu.VMEM((CH, da), jnp.float32),
            pltpu.VMEM_SHARED((n_pad, da), jnp.float32),
        ],
        compiler_params=pltpu.CompilerParams(use_tc_tiling_on_sc=False),
    )
    def sc_k(haug_hbm, e_hbm, z_hbm, out_hbm, idx_v, rows_v, acc_sh):
        c = lax.axis_index("c")
        s = lax.axis_index("s")
        wid = c * NS + s
        r0 = s * rpt
        pltpu.sync_copy(z_hbm.at[pl.ds(r0, rpt)], acc_sh.at[pl.ds(r0, rpt)])
        plsc.subcore_barrier()
        base = wid * epw

        @pl.loop(0, k_chunks)
        def _(k):
            off = base + k * CH
            pltpu.sync_copy(e_hbm.at[0, pl.ds(off, CH)], idx_v.at[0])
            pltpu.sync_copy(e_hbm.at[1, pl.ds(off, CH)], idx_v.at[1])
            pltpu.sync_copy(haug_hbm.at[idx_v.at[0]], rows_v)
            pltpu.sync_copy(rows_v, acc_sh.at[idx_v.at[1]], add=True)

        plsc.subcore_barrier()
        pltpu.sync_copy(acc_sh.at[pl.ds(r0, rpt)], out_hbm.at[c, pl.ds(r0, rpt)])

    return sc_k(h_aug, e, z)


def _tc_finish(acc, h, w, b2):
    n, d = h.shape
    da = acc.shape[2]
    d_out = w.shape[1]
    blk = 1000 if n % 1000 == 0 else 8
    grid = n // blk

    def body(acc_ref, h_ref, w_ref, b_ref, o_ref):
        p = acc_ref[0] + acc_ref[1]
        sums = p[:, :d]
        cnt = p[:, d:d + 1]
        h_n = sums / jnp.maximum(cnt, 1.0)
        o_ref[...] = (
            jnp.dot(h_ref[...], w_ref[:d, :], preferred_element_type=jnp.float32)
            + jnp.dot(h_n, w_ref[d:, :], preferred_element_type=jnp.float32)
            + b_ref[...]
        )

    return pl.pallas_call(
        body,
        grid=(grid,),
        in_specs=[
            pl.BlockSpec((2, blk, da), lambda i: (0, i, 0)),
            pl.BlockSpec((blk, d), lambda i: (i, 0)),
            pl.BlockSpec((2 * d, d_out), lambda i: (0, 0)),
            pl.BlockSpec((1, d_out), lambda i: (0, 0)),
        ],
        out_specs=pl.BlockSpec((blk, d_out), lambda i: (i, 0)),
        out_shape=jax.ShapeDtypeStruct((n, d_out), jnp.float32),
    )(acc, h, w, b2)


def kernel(h, edge_index, W, b):
    n, d = h.shape
    e_cnt = edge_index.shape[1]
    da = ((d + 1 + 15) // 16) * 16
    n_pad = ((n + 1 + NS * 8 - 1) // (NS * 8)) * (NS * 8)
    step = NC * NS * CH
    e_pad_cnt = ((e_cnt + step - 1) // step) * step

    e32 = edge_index.astype(jnp.int32)
    if e_pad_cnt != e_cnt:
        pad = jnp.full((2, e_pad_cnt - e_cnt), n, jnp.int32)
        e32 = jnp.concatenate([e32, pad], axis=1)
    h_aug = jnp.zeros((n_pad, da), jnp.float32)
    h_aug = h_aug.at[:n, :d].set(h).at[:n, d].set(1.0)
    z = jnp.zeros((n_pad, da), jnp.float32)

    acc = _sc_aggregate(h_aug, e32, z)
    return _tc_finish(acc[:, :n, :], h, W, b.reshape(1, -1))
